# Initial kernel scaffold; baseline (speedup 1.0000x reference)
import jax
import jax.numpy as jnp
from jax.experimental import pallas as pl


def kernel(x, ew0, eb0, ew1, eb1, ew2, eb2, er0w1, er0b1, er0w2, er0b2, er1w1, er1b1, er1w2, er1b2, epw, epb, codebook, dr0w1, dr0b1, dr0w2, dr0b2, dr1w1, dr1b1, dr1w2, dr1b2, dt0w, dt0b, dt1w, dt1b, dt2w, dt2b):
    raise NotImplementedError("write your pallas kernel here")


# speedup vs baseline: 1.0431x; 1.0431x over previous
import functools

import jax
import jax.numpy as jnp
import numpy as np
from jax import lax
from jax.experimental import pallas as pl
from jax.experimental.pallas import tpu as pltpu

K_CODES = 512
D_DIM = 256
ROW_BLOCK = 784


def _conv(x, w, b, stride, pad):
    out = lax.conv_general_dilated(
        x, w, (stride, stride), ((pad, pad), (pad, pad)),
        dimension_numbers=('NCHW', 'OIHW', 'NCHW'))
    return out + b[None, :, None, None]


def _conv_t(x, w, b):
    w2 = jnp.flip(w, axis=(2, 3)).transpose(1, 0, 2, 3)
    out = lax.conv_general_dilated(
        x, w2, (1, 1), ((2, 2), (2, 2)), lhs_dilation=(2, 2),
        dimension_numbers=('NCHW', 'OIHW', 'NCHW'))
    return out + b[None, :, None, None]


def _resblock(x, w1, b1, w2, b2):
    h = jax.nn.relu(x)
    h = _conv(h, w1, b1, 1, 1)
    h = jax.nn.relu(h)
    h = _conv(h, w2, b2, 1, 1)
    return h + x


def _vq_kernel(ze_ref, cb_ref, zq_ref, idx_ref, loss_ref):
    step = pl.program_id(0)
    z = ze_ref[...]
    n1 = z / (jnp.sqrt(jnp.sum(z * z, axis=1, keepdims=True)) + 1e-12)
    n2 = n1 / (jnp.sqrt(jnp.sum(n1 * n1, axis=1, keepdims=True)) + 1e-12)
    cb = cb_ref[...]
    cbn = cb / (jnp.sqrt(jnp.sum(cb * cb, axis=1, keepdims=True)) + 1e-12)
    sim = jax.lax.dot_general(
        n2, cbn, (((1,), (1,)), ((), ())), preferred_element_type=jnp.float32)
    m = jnp.max(sim, axis=1, keepdims=True)
    cols = lax.broadcasted_iota(jnp.int32, sim.shape, 1)
    idx = jnp.min(jnp.where(sim == m, cols, K_CODES), axis=1, keepdims=True)
    onehot = (cols == idx).astype(jnp.float32)
    zq = jax.lax.dot_general(
        onehot, cbn, (((1,), (0,)), ((), ())), preferred_element_type=jnp.float32)
    zq_ref[...] = zq
    idx_ref[...] = idx

    @pl.when(step == 0)
    def _():
        loss_ref[...] = jnp.zeros_like(loss_ref)

    d = n1 - zq
    loss_ref[...] += jnp.sum(d * d).reshape(1, 1)


def _vq(ze_flat, codebook):
    n = ze_flat.shape[0]
    grid = n // ROW_BLOCK
    zq, idx, losssum = pl.pallas_call(
        _vq_kernel,
        grid=(grid,),
        in_specs=[
            pl.BlockSpec((ROW_BLOCK, D_DIM), lambda i: (i, 0)),
            pl.BlockSpec((K_CODES, D_DIM), lambda i: (0, 0)),
        ],
        out_specs=[
            pl.BlockSpec((ROW_BLOCK, D_DIM), lambda i: (i, 0)),
            pl.BlockSpec((ROW_BLOCK, 1), lambda i: (i, 0)),
            pl.BlockSpec((1, 1), lambda i: (0, 0)),
        ],
        out_shape=[
            jax.ShapeDtypeStruct((n, D_DIM), jnp.float32),
            jax.ShapeDtypeStruct((n, 1), jnp.int32),
            jax.ShapeDtypeStruct((1, 1), jnp.float32),
        ],
    )(ze_flat, codebook)
    return zq, idx[:, 0], losssum[0, 0]


def kernel(x, ew0, eb0, ew1, eb1, ew2, eb2, er0w1, er0b1, er0w2, er0b2,
           er1w1, er1b1, er1w2, er1b2, epw, epb, codebook,
           dr0w1, dr0b1, dr0w2, dr0b2, dr1w1, dr1b1, dr1w2, dr1b2,
           dt0w, dt0b, dt1w, dt1b, dt2w, dt2b):
    h = jax.nn.relu(_conv(x, ew0, eb0, 2, 1))
    h = jax.nn.relu(_conv(h, ew1, eb1, 2, 1))
    h = jax.nn.relu(_conv(h, ew2, eb2, 2, 1))
    h = _resblock(h, er0w1, er0b1, er0w2, er0b2)
    h = _resblock(h, er1w1, er1b1, er1w2, er1b2)
    z_e = _conv(h, epw, epb, 1, 0)
    B, D, Hh, Ww = z_e.shape
    ze_flat = z_e.transpose(0, 2, 3, 1).reshape(-1, D)
    zq_flat, idx, losssum = _vq(ze_flat, codebook)
    commitment_loss = 0.25 * losssum / (B * D * Hh * Ww)
    z_q = zq_flat.reshape(B, Hh, Ww, D).transpose(0, 3, 1, 2)
    h = _resblock(z_q, dr0w1, dr0b1, dr0w2, dr0b2)
    h = _resblock(h, dr1w1, dr1b1, dr1w2, dr1b2)
    h = jax.nn.relu(_conv_t(h, dt0w, dt0b))
    h = jax.nn.relu(_conv_t(h, dt1w, dt1b))
    recon = jax.nn.sigmoid(_conv_t(h, dt2w, dt2b))
    return recon, idx.reshape(B, Hh, Ww), commitment_loss
